# baseline (speedup 1.0000x reference)
import functools

import jax
import jax.numpy as jnp
from jax import lax
from jax.experimental import pallas as pl
from jax.experimental.pallas import tpu as pltpu
from jax.experimental.pallas import tpu_sc as plsc

B = 16384
F = 64
NC = 2
NS = 16
NW = NC * NS
BPW = B // NW
CH = 128
NCH = BPW // CH


def _gmf_body(user_h, item_h, eu_h, ei_h, wb_h, out_h,
              uix, iix, eu_v, ei_v, wb_v, out_v, tmp_v, sem):
    wid = lax.axis_index("s") * NC + lax.axis_index("c")
    base = wid * BPW

    pltpu.sync_copy(user_h.at[pl.ds(base, BPW)], uix)
    pltpu.sync_copy(item_h.at[pl.ds(base, BPW)], iix)
    pltpu.sync_copy(wb_h, wb_v)

    copies = []
    for j in range(NCH):
        copies.append(pltpu.async_copy(
            eu_h.at[uix.at[pl.ds(j * CH, CH)]],
            eu_v.at[pl.ds(j * CH, CH)], sem))
        copies.append(pltpu.async_copy(
            ei_h.at[iix.at[pl.ds(j * CH, CH)]],
            ei_v.at[pl.ds(j * CH, CH)], sem))
    for c in copies:
        c.wait()

    w0 = wb_v[pl.ds(0, 16)]
    w1 = wb_v[pl.ds(16, 16)]
    w2 = wb_v[pl.ds(32, 16)]
    w3 = wb_v[pl.ds(48, 16)]
    bias = wb_v[pl.ds(F, 16)]
    row_iota = lax.iota(jnp.int32, 16)

    def body(blk, carry):
        for i in range(16):
            r = blk * 16 + i
            acc = (eu_v[r, pl.ds(0, 16)] * ei_v[r, pl.ds(0, 16)] * w0
                   + eu_v[r, pl.ds(16, 16)] * ei_v[r, pl.ds(16, 16)] * w1
                   + eu_v[r, pl.ds(32, 16)] * ei_v[r, pl.ds(32, 16)] * w2
                   + eu_v[r, pl.ds(48, 16)] * ei_v[r, pl.ds(48, 16)] * w3)
            tmp_v[i, pl.ds(0, 16)] = acc
        sums = bias
        for c in range(16):
            col = jnp.full((16,), c, jnp.int32)
            sums = sums + plsc.load_gather(tmp_v, [row_iota, col])
        out_v[pl.ds(blk * 16, 16)] = sums
        return carry

    lax.fori_loop(0, BPW // 16, body, 0)

    pltpu.sync_copy(out_v, out_h.at[pl.ds(base, BPW)])


def kernel(user, item, embed_user, embed_item, W, b):
    wb = jnp.zeros((128,), jnp.float32)
    wb = wb.at[0:F].set(W.reshape(-1))
    wb = wb.at[F:F + 16].set(jnp.broadcast_to(b, (16,)))

    mesh = plsc.VectorSubcoreMesh(core_axis_name="c", subcore_axis_name="s")
    run = functools.partial(
        pl.kernel,
        mesh=mesh,
        compiler_params=pltpu.CompilerParams(
            needs_layout_passes=False, use_tc_tiling_on_sc=False),
        out_type=jax.ShapeDtypeStruct((B,), jnp.float32),
        scratch_types=[
            pltpu.VMEM((BPW,), jnp.int32),
            pltpu.VMEM((BPW,), jnp.int32),
            pltpu.VMEM((BPW, F), jnp.float32),
            pltpu.VMEM((BPW, F), jnp.float32),
            pltpu.VMEM((128,), jnp.float32),
            pltpu.VMEM((BPW,), jnp.float32),
            pltpu.VMEM((16, 17), jnp.float32),
            pltpu.SemaphoreType.DMA,
        ],
    )(_gmf_body)
    return run(user, item, embed_user, embed_item, wb)


# speedup vs baseline: 1.0015x; 1.0015x over previous
import functools

import jax
import jax.numpy as jnp
from jax import lax
from jax.experimental import pallas as pl
from jax.experimental.pallas import tpu as pltpu
from jax.experimental.pallas import tpu_sc as plsc

B = 16384
F = 64
NC = 2
NS = 16
NW = NC * NS
BPW = B // NW
CH = 128
NCH = BPW // CH
NBLK = BPW // 16
BPC = CH // 16


def _tree_sum(vals):
    vals = list(vals)
    while len(vals) > 1:
        nxt = [a + b for a, b in zip(vals[0::2], vals[1::2])]
        if len(vals) % 2:
            nxt.append(vals[-1])
        vals = nxt
    return vals[0]


def _gmf_body(user_h, item_h, eu_h, ei_h, wb_h, out_h,
              uix, iix, eu_v, ei_v, wb_v, out_v, tmp_v, sems):
    wid = lax.axis_index("s") * NC + lax.axis_index("c")
    base = wid * BPW

    pltpu.sync_copy(user_h.at[pl.ds(base, BPW)], uix)
    pltpu.sync_copy(item_h.at[pl.ds(base, BPW)], iix)
    pltpu.sync_copy(wb_h, wb_v)

    handles = []
    for j in range(NCH):
        sem = sems.at[j]
        h1 = pltpu.async_copy(
            eu_h.at[uix.at[pl.ds(j * CH, CH)]],
            eu_v.at[pl.ds(j * CH, CH)], sem)
        h2 = pltpu.async_copy(
            ei_h.at[iix.at[pl.ds(j * CH, CH)]],
            ei_v.at[pl.ds(j * CH, CH)], sem)
        handles.append((h1, h2))

    w0 = wb_v[pl.ds(0, 16)]
    w1 = wb_v[pl.ds(16, 16)]
    w2 = wb_v[pl.ds(32, 16)]
    w3 = wb_v[pl.ds(48, 16)]
    bias = wb_v[pl.ds(F, 16)]
    row_iota = lax.iota(jnp.int32, 16)

    def block(blk, carry):
        for i in range(16):
            r = blk * 16 + i
            acc = _tree_sum([
                eu_v[r, pl.ds(0, 16)] * ei_v[r, pl.ds(0, 16)] * w0,
                eu_v[r, pl.ds(16, 16)] * ei_v[r, pl.ds(16, 16)] * w1,
                eu_v[r, pl.ds(32, 16)] * ei_v[r, pl.ds(32, 16)] * w2,
                eu_v[r, pl.ds(48, 16)] * ei_v[r, pl.ds(48, 16)] * w3,
            ])
            tmp_v[blk, i, pl.ds(0, 16)] = acc
        blk_idx = jnp.full((16,), blk, jnp.int32)
        cols = [
            plsc.load_gather(tmp_v,
                             [blk_idx, row_iota, jnp.full((16,), c, jnp.int32)])
            for c in range(16)
        ]
        out_v[pl.ds(blk * 16, 16)] = _tree_sum(cols) + bias
        return carry

    for j in range(NCH):
        h1, h2 = handles[j]
        h1.wait()
        h2.wait()
        lax.fori_loop(j * BPC, (j + 1) * BPC, block, 0)

    pltpu.sync_copy(out_v, out_h.at[pl.ds(base, BPW)])


def kernel(user, item, embed_user, embed_item, W, b):
    wb = jnp.concatenate([
        W.reshape(-1),
        jnp.broadcast_to(b, (16,)),
        jnp.zeros((48,), jnp.float32),
    ])

    mesh = plsc.VectorSubcoreMesh(core_axis_name="c", subcore_axis_name="s")
    run = functools.partial(
        pl.kernel,
        mesh=mesh,
        compiler_params=pltpu.CompilerParams(
            needs_layout_passes=False, use_tc_tiling_on_sc=False),
        out_type=jax.ShapeDtypeStruct((B,), jnp.float32),
        scratch_types=[
            pltpu.VMEM((BPW,), jnp.int32),
            pltpu.VMEM((BPW,), jnp.int32),
            pltpu.VMEM((BPW, F), jnp.float32),
            pltpu.VMEM((BPW, F), jnp.float32),
            pltpu.VMEM((128,), jnp.float32),
            pltpu.VMEM((BPW,), jnp.float32),
            pltpu.VMEM((NBLK, 16, 17), jnp.float32),
            pltpu.SemaphoreType.DMA((NCH,)),
        ],
    )(_gmf_body)
    return run(user, item, embed_user, embed_item, wb)
